# baseline (speedup 1.0000x reference)
import functools

import jax
import jax.numpy as jnp
from jax.experimental import pallas as pl
from jax.experimental.pallas import tpu as pltpu

B, L, D = 4, 2048, 1024
KD, VD, S, H, TOPK = 64, 256, 100000, 256, 8

SC_CHUNK = 8192
NSC = (S + SC_CHUNK - 1) // SC_CHUNK
S_PAD = NSC * SC_CHUNK
CK2 = SC_CHUNK // 2
LAST2_BASE = (NSC - 1) * CK2
LAST2_ROWS = S // 2 - LAST2_BASE
LTQ = 512
NLTQ = L // LTQ
LT = 512
NLT = L // LT


def _main_body(x_ref, wk_ref, bk_ref, keys_hbm, tv_ref, ti_ref,
               acc_ref, q_ref, kbuf_ref, sims_ref, ksem):
    j = pl.program_id(0)

    @pl.when(j == 0)
    def _():
        for c in range(NSC - 1):
            pltpu.make_async_copy(
                keys_hbm.at[pl.ds(c * CK2, CK2)],
                kbuf_ref.at[c],
                ksem.at[c],
            ).start()
        pltpu.make_async_copy(
            keys_hbm.at[pl.ds(LAST2_BASE, LAST2_ROWS)],
            kbuf_ref.at[NSC - 1, pl.ds(0, LAST2_ROWS)],
            ksem.at[NSC - 1],
        ).start()

    @pl.when(j < NLTQ)
    def _():
        part = jnp.sum(x_ref[...], axis=1)

        @pl.when(j == 0)
        def _():
            acc_ref[...] = part

        @pl.when(j > 0)
        def _():
            acc_ref[...] = acc_ref[...] + part

        @pl.when(j == NLTQ - 1)
        def _():
            xbar = acc_ref[...] * (1.0 / L)
            q_ref[0:B] = (
                jnp.dot(xbar, wk_ref[...], preferred_element_type=jnp.float32)
                + bk_ref[...]
            )

    @pl.when(jnp.logical_and(j >= NLTQ, j < NLTQ + NSC))
    def _():
        c = j - NLTQ

        @pl.when(c < NSC - 1)
        def _():
            pltpu.make_async_copy(
                keys_hbm.at[pl.ds(c * CK2, CK2)],
                kbuf_ref.at[c],
                ksem.at[c],
            ).wait()

        @pl.when(c == NSC - 1)
        def _():
            pltpu.make_async_copy(
                keys_hbm.at[pl.ds(LAST2_BASE, LAST2_ROWS)],
                kbuf_ref.at[NSC - 1, pl.ds(0, LAST2_ROWS)],
                ksem.at[NSC - 1],
            ).wait()

        kc = kbuf_ref[c]
        qv = q_ref[0:B]
        se = jax.lax.dot_general(
            qv, kc[:, 0:KD], (((1,), (1,)), ((), ())),
            preferred_element_type=jnp.float32,
        )
        so = jax.lax.dot_general(
            qv, kc[:, KD:128], (((1,), (1,)), ((), ())),
            preferred_element_type=jnp.float32,
        )

        @pl.when(c == NSC - 1)
        def _():
            col = jax.lax.broadcasted_iota(jnp.int32, (B, CK2), 1)
            sims_ref[:, 2 * c] = jnp.where(col < LAST2_ROWS, se, -1e30)
            sims_ref[:, 2 * c + 1] = jnp.where(col < LAST2_ROWS, so, -1e30)

        @pl.when(c < NSC - 1)
        def _():
            sims_ref[:, 2 * c] = se
            sims_ref[:, 2 * c + 1] = so

    @pl.when(j >= NLTQ + NSC)
    def _():
        b2 = jnp.maximum(j - (NLTQ + NSC), 0)
        s = sims_ref[b2]
        r0 = jax.lax.broadcasted_iota(jnp.int32, (2 * NSC, CK2), 0)
        t0 = jax.lax.broadcasted_iota(jnp.int32, (2 * NSC, CK2), 1)
        idx = (r0 // 2) * SC_CHUNK + (r0 % 2) + 2 * t0
        lane = jax.lax.broadcasted_iota(jnp.int32, (1, 1, 128), 2)
        tv = jnp.zeros((1, 1, 128), jnp.float32)
        ti = jnp.zeros((1, 1, 128), jnp.int32)
        for k in range(TOPK):
            m = jnp.max(s)
            cand = jnp.where(s == m, idx, jnp.int32(2**31 - 1))
            fi = jnp.min(cand)
            tv = jnp.where(lane == k, m, tv)
            ti = jnp.where(lane == k, fi, ti)
            s = jnp.where(idx == fi, -3e38, s)
        tv_ref[...] = tv
        ti_ref[...] = ti


def _main_call(x, W_key, b_key2d, cube_keys):
    nsteps = NLTQ + NSC + B
    out_idx = lambda j: (jnp.clip(j - (NLTQ + NSC), 0, B - 1), 0, 0)
    return pl.pallas_call(
        _main_body,
        grid=(nsteps,),
        in_specs=[
            pl.BlockSpec((B, LTQ, D),
                         lambda j: (0, jnp.minimum(j, NLTQ - 1), 0)),
            pl.BlockSpec((D, KD), lambda j: (0, 0)),
            pl.BlockSpec((1, KD), lambda j: (0, 0)),
            pl.BlockSpec(memory_space=pl.ANY),
        ],
        out_specs=[
            pl.BlockSpec((1, 1, 128), out_idx),
            pl.BlockSpec((1, 1, 128), out_idx),
        ],
        out_shape=[
            jax.ShapeDtypeStruct((B, 1, 128), jnp.float32),
            jax.ShapeDtypeStruct((B, 1, 128), jnp.int32),
        ],
        scratch_shapes=[
            pltpu.VMEM((B, D), jnp.float32),
            pltpu.VMEM((8, KD), jnp.float32),
            pltpu.VMEM((NSC, CK2, 128), jnp.float32),
            pltpu.VMEM((B, 2 * NSC, CK2), jnp.float32),
            pltpu.SemaphoreType.DMA((NSC,)),
        ],
    )(x, W_key, b_key2d, cube_keys)


def _fused_body(x_ref, wg1_ref, bg1_ref, wrow_ref, conf_ref, mv_ref,
                wmem_ref, bmem_ref, wg2_ref, bg2_ref, lng_ref, lnb_ref,
                out_ref, mem_ref):
    b = pl.program_id(0)
    j = pl.program_id(1)

    @pl.when(jnp.logical_and(b == 0, j == 0))
    def _():
        mem_ref[0:B] = (
            jnp.dot(mv_ref[...], wmem_ref[...],
                    preferred_element_type=jnp.float32)
            + bmem_ref[...]
        )

    xt = x_ref[0]
    t = jnp.dot(xt.astype(jnp.bfloat16), wg1_ref[...],
                preferred_element_type=jnp.float32)
    tb = t + bg1_ref[...] + conf_ref[0, 0] * wrow_ref[...]
    h = 0.5 * tb * (1.0 + jax.lax.erf(tb * 0.7071067811865476))
    sv = jnp.dot(h, wg2_ref[...], preferred_element_type=jnp.float32)
    alpha = jax.nn.sigmoid(sv[:, 0:1] + bg2_ref[0, 0])
    y = xt + (1.0 - alpha) * mem_ref[pl.ds(b, 1)]
    mu = jnp.mean(y, axis=1, keepdims=True)
    var = jnp.mean((y - mu) ** 2, axis=1, keepdims=True)
    out_ref[0] = (y - mu) * jax.lax.rsqrt(var + 1e-5) * lng_ref[...] + lnb_ref[...]


def _fused_call(x, wg1a, bg1, wrow, conf2d, mem_val, W_mem, bmem2d,
                wg2p, bg2v, lng, lnb):
    zero2 = lambda b, j: (0, 0)
    return pl.pallas_call(
        _fused_body,
        grid=(B, NLT),
        in_specs=[
            pl.BlockSpec((1, LT, D), lambda b, j: (b, j, 0)),
            pl.BlockSpec((D, H), zero2),
            pl.BlockSpec((1, H), zero2),
            pl.BlockSpec((1, H), zero2),
            pl.BlockSpec((1, 1), zero2),
            pl.BlockSpec((B, VD), zero2),
            pl.BlockSpec((VD, D), zero2),
            pl.BlockSpec((1, D), zero2),
            pl.BlockSpec((H, 128), zero2),
            pl.BlockSpec((1, 1), zero2),
            pl.BlockSpec((1, D), zero2),
            pl.BlockSpec((1, D), zero2),
        ],
        out_specs=pl.BlockSpec((1, LT, D), lambda b, j: (b, j, 0)),
        out_shape=jax.ShapeDtypeStruct((B, L, D), jnp.float32),
        scratch_shapes=[pltpu.VMEM((8, D), jnp.float32)],
    )(x, wg1a, bg1, wrow, conf2d, mem_val, W_mem, bmem2d, wg2p, bg2v,
      lng, lnb)


def kernel(x, W_key, b_key, cube_keys, cube_values, W_mem, b_mem,
           Wg1, bg1, Wg2, bg2, ln_g, ln_b):
    tv, ti = _main_call(x, W_key, b_key.reshape(1, KD),
                        cube_keys.reshape(S // 2, 128))
    topv = tv[:, 0, :TOPK]
    topi = ti[:, 0, :TOPK]
    w = jax.nn.softmax(topv, axis=-1)
    conf = jnp.mean(jnp.max(w, axis=-1))
    gathered = jnp.take(cube_values, topi, axis=0)
    mem_val = jnp.sum(w[..., None] * gathered, axis=1)
    return _fused_call(
        x, Wg1[:D].astype(jnp.bfloat16), bg1.reshape(1, H),
        Wg1[D].reshape(1, H), conf.reshape(1, 1), mem_val, W_mem,
        b_mem.reshape(1, D), jnp.pad(Wg2, ((0, 0), (0, 127))),
        bg2.reshape(1, 1), ln_g.reshape(1, D), ln_b.reshape(1, D))


# speedup vs baseline: 1.0701x; 1.0701x over previous
import functools

import jax
import jax.numpy as jnp
from jax.experimental import pallas as pl
from jax.experimental.pallas import tpu as pltpu
from jax.experimental.pallas import tpu_sc as plsc

B, L, D = 4, 2048, 1024
KD, VD, S, H, TOPK = 64, 256, 100000, 256, 8

SC_CHUNK = 8192
NSC = (S + SC_CHUNK - 1) // SC_CHUNK
S_PAD = NSC * SC_CHUNK
SROWS = S_PAD // 128
LT = 256
NLT = L // LT
LTQ = 512
NLTQ = L // LTQ


def _qs_body(x_ref, wk_ref, bk_ref, keys_ref, s_ref, acc_ref, q_ref):
    j = pl.program_id(0)

    @pl.when(j < NLTQ)
    def _():
        part = jnp.sum(x_ref[...], axis=1)

        @pl.when(j == 0)
        def _():
            acc_ref[...] = part

        @pl.when(j > 0)
        def _():
            acc_ref[...] = acc_ref[...] + part

        @pl.when(j == NLTQ - 1)
        def _():
            xbar = acc_ref[...] * (1.0 / L)
            q_ref[0:B] = (
                jnp.dot(xbar, wk_ref[...], preferred_element_type=jnp.float32)
                + bk_ref[...]
            )

    @pl.when(j >= NLTQ)
    def _():
        c = j - NLTQ
        s = jax.lax.dot_general(
            q_ref[0:B], keys_ref[...], (((1,), (1,)), ((), ())),
            preferred_element_type=jnp.float32,
        )
        col = c * SC_CHUNK + jax.lax.broadcasted_iota(
            jnp.int32, (B, SC_CHUNK), 1)
        s_ref[...] = jnp.where(col < S, s, -1e30)


def _qs_call(x, W_key, b_key2d, cube_keys):
    return pl.pallas_call(
        _qs_body,
        grid=(NLTQ + NSC,),
        in_specs=[
            pl.BlockSpec((B, LTQ, D), lambda j: (0, jnp.minimum(j, NLTQ - 1), 0)),
            pl.BlockSpec((D, KD), lambda j: (0, 0)),
            pl.BlockSpec((1, KD), lambda j: (0, 0)),
            pl.BlockSpec((SC_CHUNK, KD),
                         lambda j: (jnp.maximum(j - NLTQ, 0), 0)),
        ],
        out_specs=pl.BlockSpec((B, SC_CHUNK),
                               lambda j: (0, jnp.maximum(j - NLTQ, 0))),
        out_shape=jax.ShapeDtypeStruct((B, S_PAD), jnp.float32),
        scratch_shapes=[
            pltpu.VMEM((B, D), jnp.float32),
            pltpu.VMEM((8, KD), jnp.float32),
        ],
    )(x, W_key, b_key2d, cube_keys)


def _topk_body(s_ref, tv_ref, ti_ref):
    s = s_ref[0]
    idx = (
        jax.lax.broadcasted_iota(jnp.int32, (SROWS, 128), 0) * 128
        + jax.lax.broadcasted_iota(jnp.int32, (SROWS, 128), 1)
    )
    lane = jax.lax.broadcasted_iota(jnp.int32, (1, 1, 128), 2)
    tv = jnp.zeros((1, 1, 128), jnp.float32)
    ti = jnp.zeros((1, 1, 128), jnp.int32)
    for k in range(TOPK):
        m = jnp.max(s)
        cand = jnp.where(s == m, idx, jnp.int32(2**31 - 1))
        fi = jnp.min(cand)
        tv = jnp.where(lane == k, m, tv)
        ti = jnp.where(lane == k, fi, ti)
        s = jnp.where(idx == fi, -3e38, s)
    tv_ref[...] = tv
    ti_ref[...] = ti


def _topk_call(sims3d):
    return pl.pallas_call(
        _topk_body,
        grid=(B,),
        in_specs=[pl.BlockSpec((1, SROWS, 128), lambda b: (b, 0, 0))],
        out_specs=[
            pl.BlockSpec((1, 1, 128), lambda b: (b, 0, 0)),
            pl.BlockSpec((1, 1, 128), lambda b: (b, 0, 0)),
        ],
        out_shape=[
            jax.ShapeDtypeStruct((B, 1, 128), jnp.float32),
            jax.ShapeDtypeStruct((B, 1, 128), jnp.int32),
        ],
    )(sims3d)


def _sc_tail_body(tv_hbm, ti_hbm, vals_hbm, out_mem, out_w,
                  tvv, idxv, rows_v, w_v, mem_v, sem):
    cid = jax.lax.axis_index("c")
    sid = jax.lax.axis_index("s")
    wid = cid * 16 + sid
    lane = jax.lax.iota(jnp.int32, 16)

    @pl.when(wid < B)
    def _():
        row = wid
        pltpu.sync_copy(tv_hbm.at[row], tvv)
        pltpu.sync_copy(ti_hbm.at[row], idxv)
        tv = tvv[...]
        m0 = tv[0]
        e = jnp.where(lane < TOPK, jnp.exp(tv - m0), 0.0)
        ssum = e[0] + e[1] + e[2] + e[3] + e[4] + e[5] + e[6] + e[7]
        wv = e / ssum
        w_v[...] = wv
        pltpu.sync_copy(w_v, out_w.at[row])
        pltpu.async_copy(vals_hbm.at[idxv], rows_v, sem).wait()
        ws = [wv[jj] for jj in range(TOPK)]
        for cc in range(VD // 16):
            a = ws[0] * rows_v[0, pl.ds(cc * 16, 16)]
            for jj in range(1, TOPK):
                a = a + ws[jj] * rows_v[jj, pl.ds(cc * 16, 16)]
            mem_v[pl.ds(cc * 16, 16)] = a
        pltpu.sync_copy(mem_v, out_mem.at[row])


def _sc_tail(topv16, topi16, cube_values):
    f = pl.kernel(
        _sc_tail_body,
        mesh=plsc.VectorSubcoreMesh(core_axis_name="c", subcore_axis_name="s"),
        out_type=[
            jax.ShapeDtypeStruct((B, VD), jnp.float32),
            jax.ShapeDtypeStruct((B, 16), jnp.float32),
        ],
        scratch_types=[
            pltpu.VMEM((16,), jnp.float32),
            pltpu.VMEM((16,), jnp.int32),
            pltpu.VMEM((16, VD), jnp.float32),
            pltpu.VMEM((16,), jnp.float32),
            pltpu.VMEM((VD,), jnp.float32),
            pltpu.SemaphoreType.DMA,
        ],
    )
    return f(topv16, topi16, cube_values)


def _fused_body(x_ref, wg1_ref, bg1_ref, wrow_ref, conf_ref, mv_ref,
                wmem_ref, bmem_ref, wg2_ref, bg2_ref, lng_ref, lnb_ref,
                out_ref, mem_ref):
    b = pl.program_id(0)
    j = pl.program_id(1)

    @pl.when(jnp.logical_and(b == 0, j == 0))
    def _():
        mem_ref[0:B] = (
            jnp.dot(mv_ref[...], wmem_ref[...],
                    preferred_element_type=jnp.float32)
            + bmem_ref[...]
        )

    xt = x_ref[0]
    t = jnp.dot(xt.astype(jnp.bfloat16), wg1_ref[...],
                preferred_element_type=jnp.float32)
    tb = t + bg1_ref[...] + conf_ref[0, 0] * wrow_ref[...]
    h = 0.5 * tb * (1.0 + jax.lax.erf(tb * 0.7071067811865476))
    sv = jnp.dot(h, wg2_ref[...], preferred_element_type=jnp.float32)
    alpha = jax.nn.sigmoid(sv[:, 0:1] + bg2_ref[0, 0])
    y = xt + (1.0 - alpha) * mem_ref[pl.ds(b, 1)]
    mu = jnp.mean(y, axis=1, keepdims=True)
    var = jnp.mean((y - mu) ** 2, axis=1, keepdims=True)
    out_ref[0] = (y - mu) * jax.lax.rsqrt(var + 1e-5) * lng_ref[...] + lnb_ref[...]


def _fused_call(x, wg1a, bg1, wrow, conf2d, mem_val, W_mem, bmem2d,
                wg2p, bg2v, lng, lnb):
    zero2 = lambda b, j: (0, 0)
    return pl.pallas_call(
        _fused_body,
        grid=(B, NLT),
        in_specs=[
            pl.BlockSpec((1, LT, D), lambda b, j: (b, j, 0)),
            pl.BlockSpec((D, H), zero2),
            pl.BlockSpec((1, H), zero2),
            pl.BlockSpec((1, H), zero2),
            pl.BlockSpec((1, 1), zero2),
            pl.BlockSpec((B, VD), zero2),
            pl.BlockSpec((VD, D), zero2),
            pl.BlockSpec((1, D), zero2),
            pl.BlockSpec((H, 128), zero2),
            pl.BlockSpec((1, 1), zero2),
            pl.BlockSpec((1, D), zero2),
            pl.BlockSpec((1, D), zero2),
        ],
        out_specs=pl.BlockSpec((1, LT, D), lambda b, j: (b, j, 0)),
        out_shape=jax.ShapeDtypeStruct((B, L, D), jnp.float32),
        scratch_shapes=[pltpu.VMEM((8, D), jnp.float32)],
    )(x, wg1a, bg1, wrow, conf2d, mem_val, W_mem, bmem2d, wg2p, bg2v,
      lng, lnb)


def kernel(x, W_key, b_key, cube_keys, cube_values, W_mem, b_mem,
           Wg1, bg1, Wg2, bg2, ln_g, ln_b):
    sims = _qs_call(x, W_key, b_key.reshape(1, KD), cube_keys)
    tv, ti = _topk_call(sims.reshape(B, SROWS, 128))
    mem_val, wfull = _sc_tail(tv[:, 0, :16], ti[:, 0, :16], cube_values)
    conf = jnp.mean(jnp.max(wfull, axis=-1))
    return _fused_call(
        x, Wg1[:D].astype(jnp.bfloat16), bg1.reshape(1, H),
        Wg1[D].reshape(1, H), conf.reshape(1, 1), mem_val, W_mem,
        b_mem.reshape(1, D), jnp.pad(Wg2, ((0, 0), (0, 127))),
        bg2.reshape(1, 1), ln_g.reshape(1, D), ln_b.reshape(1, D))


# speedup vs baseline: 1.0736x; 1.0033x over previous
import functools

import jax
import jax.numpy as jnp
from jax.experimental import pallas as pl
from jax.experimental.pallas import tpu as pltpu
from jax.experimental.pallas import tpu_sc as plsc

B, L, D = 4, 2048, 1024
KD, VD, S, H, TOPK = 64, 256, 100000, 256, 8

SC_CHUNK = 8192
NSC = (S + SC_CHUNK - 1) // SC_CHUNK
S_PAD = NSC * SC_CHUNK
SROWS = S_PAD // 128
LT = 256
NLT = L // LT
LTQ = 512
NLTQ = L // LTQ


def _qs_body(x_ref, wk_ref, bk_ref, keys_ref, s_ref, acc_ref, q_ref):
    j = pl.program_id(0)

    @pl.when(j < NLTQ)
    def _():
        part = jnp.sum(x_ref[...], axis=1)

        @pl.when(j == 0)
        def _():
            acc_ref[...] = part

        @pl.when(j > 0)
        def _():
            acc_ref[...] = acc_ref[...] + part

        @pl.when(j == NLTQ - 1)
        def _():
            xbar = acc_ref[...] * (1.0 / L)
            q_ref[0:B] = (
                jnp.dot(xbar, wk_ref[...], preferred_element_type=jnp.float32)
                + bk_ref[...]
            )

    @pl.when(j >= NLTQ)
    def _():
        c = j - NLTQ
        s = jax.lax.dot_general(
            q_ref[0:B], keys_ref[...], (((1,), (1,)), ((), ())),
            preferred_element_type=jnp.float32,
        )
        col = c * SC_CHUNK + jax.lax.broadcasted_iota(
            jnp.int32, (B, SC_CHUNK), 1)
        s_ref[...] = jnp.where(col < S, s, -1e30)


def _qs_call(x, W_key, b_key2d, cube_keys):
    return pl.pallas_call(
        _qs_body,
        grid=(NLTQ + NSC,),
        in_specs=[
            pl.BlockSpec((B, LTQ, D), lambda j: (0, jnp.minimum(j, NLTQ - 1), 0)),
            pl.BlockSpec((D, KD), lambda j: (0, 0)),
            pl.BlockSpec((1, KD), lambda j: (0, 0)),
            pl.BlockSpec((SC_CHUNK, KD),
                         lambda j: (jnp.maximum(j - NLTQ, 0), 0)),
        ],
        out_specs=pl.BlockSpec((B, SC_CHUNK),
                               lambda j: (0, jnp.maximum(j - NLTQ, 0))),
        out_shape=jax.ShapeDtypeStruct((B, S_PAD), jnp.float32),
        scratch_shapes=[
            pltpu.VMEM((B, D), jnp.float32),
            pltpu.VMEM((8, KD), jnp.float32),
        ],
    )(x, W_key, b_key2d, cube_keys)


def _topk_body(s_ref, tv_ref, ti_ref):
    s = s_ref[0]
    idx = (
        jax.lax.broadcasted_iota(jnp.int32, (SROWS, 128), 0) * 128
        + jax.lax.broadcasted_iota(jnp.int32, (SROWS, 128), 1)
    )
    lane = jax.lax.broadcasted_iota(jnp.int32, (1, 1, 128), 2)
    tv = jnp.zeros((1, 1, 128), jnp.float32)
    ti = jnp.zeros((1, 1, 128), jnp.int32)
    for k in range(TOPK):
        m = jnp.max(s)
        cand = jnp.where(s == m, idx, jnp.int32(2**31 - 1))
        fi = jnp.min(cand)
        tv = jnp.where(lane == k, m, tv)
        ti = jnp.where(lane == k, fi, ti)
        s = jnp.where(idx == fi, -3e38, s)
    tv_ref[...] = tv
    ti_ref[...] = ti


def _topk_call(sims3d):
    return pl.pallas_call(
        _topk_body,
        grid=(B,),
        in_specs=[pl.BlockSpec((1, SROWS, 128), lambda b: (b, 0, 0))],
        out_specs=[
            pl.BlockSpec((1, 1, 128), lambda b: (b, 0, 0)),
            pl.BlockSpec((1, 1, 128), lambda b: (b, 0, 0)),
        ],
        out_shape=[
            jax.ShapeDtypeStruct((B, 1, 128), jnp.float32),
            jax.ShapeDtypeStruct((B, 1, 128), jnp.int32),
        ],
    )(sims3d)


def _sc_tail_body(tv_hbm, ti_hbm, vals_hbm, out_mem, out_w,
                  tvv, idxv, rows_v, w_v, mem_v, sem):
    cid = jax.lax.axis_index("c")
    sid = jax.lax.axis_index("s")
    wid = cid * 16 + sid
    lane = jax.lax.iota(jnp.int32, 16)

    @pl.when(wid < B)
    def _():
        row = wid
        pltpu.sync_copy(tv_hbm.at[row, 0, pl.ds(0, 16)], tvv)
        pltpu.sync_copy(ti_hbm.at[row, 0, pl.ds(0, 16)], idxv)
        tv = tvv[...]
        m0 = tv[0]
        e = jnp.where(lane < TOPK, jnp.exp(tv - m0), 0.0)
        ssum = e[0] + e[1] + e[2] + e[3] + e[4] + e[5] + e[6] + e[7]
        wv = e / ssum
        w_v[...] = wv
        pltpu.sync_copy(w_v, out_w.at[row])
        pltpu.async_copy(vals_hbm.at[idxv], rows_v, sem).wait()
        ws = [wv[jj] for jj in range(TOPK)]
        for cc in range(VD // 16):
            a = ws[0] * rows_v[0, pl.ds(cc * 16, 16)]
            for jj in range(1, TOPK):
                a = a + ws[jj] * rows_v[jj, pl.ds(cc * 16, 16)]
            mem_v[pl.ds(cc * 16, 16)] = a
        pltpu.sync_copy(mem_v, out_mem.at[row])


def _sc_tail(topv16, topi16, cube_values):
    f = pl.kernel(
        _sc_tail_body,
        mesh=plsc.VectorSubcoreMesh(core_axis_name="c", subcore_axis_name="s"),
        out_type=[
            jax.ShapeDtypeStruct((B, VD), jnp.float32),
            jax.ShapeDtypeStruct((B, 16), jnp.float32),
        ],
        scratch_types=[
            pltpu.VMEM((16,), jnp.float32),
            pltpu.VMEM((16,), jnp.int32),
            pltpu.VMEM((16, VD), jnp.float32),
            pltpu.VMEM((16,), jnp.float32),
            pltpu.VMEM((VD,), jnp.float32),
            pltpu.SemaphoreType.DMA,
        ],
    )
    return f(topv16, topi16, cube_values)


def _fused_body(x_ref, wg1_ref, bg1_ref, wrow_ref, conf_ref, mv_ref,
                wmem_ref, bmem_ref, wg2_ref, bg2_ref, lng_ref, lnb_ref,
                out_ref, mem_ref):
    b = pl.program_id(0)
    j = pl.program_id(1)

    @pl.when(jnp.logical_and(b == 0, j == 0))
    def _():
        mem_ref[0:B] = (
            jnp.dot(mv_ref[...], wmem_ref[...],
                    preferred_element_type=jnp.float32)
            + bmem_ref[...]
        )

    xt = x_ref[0]
    t = jnp.dot(xt.astype(jnp.bfloat16), wg1_ref[...],
                preferred_element_type=jnp.float32)
    tb = t + bg1_ref[...] + conf_ref[0, 0] * wrow_ref[...]
    h = 0.5 * tb * (1.0 + jax.lax.erf(tb * 0.7071067811865476))
    sv = jnp.dot(h, wg2_ref[...], preferred_element_type=jnp.float32)
    alpha = jax.nn.sigmoid(sv[:, 0:1] + bg2_ref[0, 0])
    y = xt + (1.0 - alpha) * mem_ref[pl.ds(b, 1)]
    mu = jnp.mean(y, axis=1, keepdims=True)
    var = jnp.mean((y - mu) ** 2, axis=1, keepdims=True)
    out_ref[0] = (y - mu) * jax.lax.rsqrt(var + 1e-5) * lng_ref[...] + lnb_ref[...]


def _fused_call(x, wg1a, bg1, wrow, conf2d, mem_val, W_mem, bmem2d,
                wg2p, bg2v, lng, lnb):
    zero2 = lambda b, j: (0, 0)
    return pl.pallas_call(
        _fused_body,
        grid=(B, NLT),
        in_specs=[
            pl.BlockSpec((1, LT, D), lambda b, j: (b, j, 0)),
            pl.BlockSpec((D, H), zero2),
            pl.BlockSpec((1, H), zero2),
            pl.BlockSpec((1, H), zero2),
            pl.BlockSpec((1, 1), zero2),
            pl.BlockSpec((B, VD), zero2),
            pl.BlockSpec((VD, D), zero2),
            pl.BlockSpec((1, D), zero2),
            pl.BlockSpec((H, 128), zero2),
            pl.BlockSpec((1, 1), zero2),
            pl.BlockSpec((1, D), zero2),
            pl.BlockSpec((1, D), zero2),
        ],
        out_specs=pl.BlockSpec((1, LT, D), lambda b, j: (b, j, 0)),
        out_shape=jax.ShapeDtypeStruct((B, L, D), jnp.float32),
        scratch_shapes=[pltpu.VMEM((8, D), jnp.float32)],
    )(x, wg1a, bg1, wrow, conf2d, mem_val, W_mem, bmem2d, wg2p, bg2v,
      lng, lnb)


def kernel(x, W_key, b_key, cube_keys, cube_values, W_mem, b_mem,
           Wg1, bg1, Wg2, bg2, ln_g, ln_b):
    sims = _qs_call(x, W_key, b_key.reshape(1, KD), cube_keys)
    tv, ti = _topk_call(sims.reshape(B, SROWS, 128))
    mem_val, wfull = _sc_tail(tv, ti, cube_values)
    conf = jnp.mean(jnp.max(wfull, axis=-1))
    return _fused_call(
        x, Wg1[:D].astype(jnp.bfloat16), bg1.reshape(1, H),
        Wg1[D].reshape(1, H), conf.reshape(1, 1), mem_val, W_mem,
        b_mem.reshape(1, D), jnp.pad(Wg2, ((0, 0), (0, 127))),
        bg2.reshape(1, 1), ln_g.reshape(1, D), ln_b.reshape(1, D))
